# Initial kernel scaffold; baseline (speedup 1.0000x reference)
import jax
import jax.numpy as jnp
from jax.experimental import pallas as pl


def kernel(context, target, emb, node_w, paths, signs, mask):
    raise NotImplementedError("write your pallas kernel here")


# speedup vs baseline: 1.6127x; 1.6127x over previous
import functools

import jax
import jax.numpy as jnp
from jax import lax
from jax.experimental import pallas as pl
from jax.experimental.pallas import tpu as pltpu
from jax.experimental.pallas import tpu_sc as plsc

NC = 2
NS = 16
NW = NC * NS
LANES = 16


def _sc_logits(context, target, emb, node_w, paths, enc, K):
    B = context.shape[0]
    V, D = emb.shape
    KP = paths.shape[1]
    NCH = D // LANES
    assert B % NW == 0 and D % LANES == 0
    C = B // NW
    GCH = 128
    assert C % GCH == 0
    IPT = 4
    assert C % (2 * IPT) == 0
    TILES = C // IPT
    NIDX = IPT * K
    assert NIDX <= 128
    TSTR = (NIDX + 7) // 8 * 8
    Q = C * K
    assert Q % LANES == 0
    NGRP = (NIDX + LANES - 1) // LANES
    TAIL = NIDX - (NGRP - 1) * LANES

    mesh = plsc.VectorSubcoreMesh(core_axis_name="c", subcore_axis_name="s")

    @functools.partial(
        pl.kernel,
        out_type=jax.ShapeDtypeStruct((B, K), jnp.float32),
        mesh=mesh,
        scratch_types=[
            pltpu.VMEM((C,), jnp.int32),
            pltpu.VMEM((C,), jnp.int32),
            pltpu.VMEM((C, D), jnp.float32),
            pltpu.VMEM((C, KP), jnp.int32),
            pltpu.VMEM((C, KP), jnp.float32),
            pltpu.VMEM((TILES, TSTR), jnp.int32),
            pltpu.VMEM((2, TSTR, D), jnp.float32),
            pltpu.VMEM((C, K), jnp.float32),
            pltpu.SemaphoreType.DMA,
            pltpu.SemaphoreType.DMA,
            pltpu.SemaphoreType.DMA,
        ],
        compiler_params=pltpu.CompilerParams(
            use_tc_tiling_on_sc=False,
            needs_layout_passes=False,
        ),
    )
    def sc_kernel(ctx_h, tgt_h, emb_h, nw_h, p_h, e_h, z_h,
                  ctx_v, tgt_v, inp_v, p_v, e_v, idx_b, w_r, z_v,
                  gsem, rs0, rs1):
        wid = lax.axis_index("s") * NC + lax.axis_index("c")
        base = wid * C
        pltpu.sync_copy(ctx_h.at[pl.ds(base, C)], ctx_v)
        pltpu.sync_copy(tgt_h.at[pl.ds(base, C)], tgt_v)

        lane = jnp.arange(LANES, dtype=jnp.int32)

        for g in range(C // GCH):
            sl = pl.ds(g * GCH, GCH)
            pltpu.async_copy(p_h.at[tgt_v.at[sl]], p_v.at[sl], gsem).wait()
            pltpu.async_copy(emb_h.at[ctx_v.at[sl]], inp_v.at[sl], gsem).wait()
            pltpu.async_copy(e_h.at[tgt_v.at[sl]], e_v.at[sl], gsem).wait()

        def flat_body(v, carry):
            jj = v * LANES + lane
            bb = jj // K
            kk = jj - bb * K
            vals = plsc.load_gather(p_v, [bb, kk])
            tt = jj // NIDX
            off = jj - tt * NIDX
            plsc.store_scatter(idx_b, [tt, off], vals)
            return carry

        NPAD = TSTR - NIDX
        zeros16 = jnp.zeros((LANES,), jnp.int32)

        def pad_body(v, carry):
            pp = v * LANES + lane
            tt = pp // NPAD
            rr = pp - tt * NPAD
            plsc.store_scatter(idx_b, [tt, NIDX + rr], zeros16)
            return carry

        lax.fori_loop(0, Q // LANES, flat_body, 0)
        lax.fori_loop(0, TILES * NPAD // LANES, pad_body, 0)

        def start(t, buf, sem):
            return pltpu.async_copy(
                nw_h.at[idx_b.at[t]], w_r.at[buf], sem)

        def compute_tile(t, half):
            zgs = [jnp.zeros((LANES,), jnp.float32) for _ in range(NGRP)]
            for ii in range(IPT):
                b_loc = t * IPT + ii
                inps = [inp_v[b_loc, pl.ds(c * LANES, LANES)] for c in range(NCH)]
                for kk in range(K):
                    jl = ii * K + kk
                    acc = w_r[half, jl, pl.ds(0, LANES)] * inps[0]
                    for c in range(1, NCH):
                        acc = acc + w_r[half, jl, pl.ds(c * LANES, LANES)] * inps[c]
                    rsum = jnp.sum(acc)
                    gi, pos = jl // LANES, jl % LANES
                    zgs[gi] = jnp.where(lane == pos, rsum, zgs[gi])
            for gi in range(NGRP):
                jj = t * NIDX + gi * LANES + lane
                jj = jnp.minimum(jj, Q - 1)
                bv = jj // K
                kv = jj - bv * K
                msk = (lane < TAIL) if gi == NGRP - 1 else None
                e16 = plsc.load_gather(e_v, [bv, kv], mask=msk)
                s16 = jnp.where(e16 > 2.0, 0.0, e16)
                moff = jnp.where(e16 > 2.0, 30.0, 0.0)
                zfin = zgs[gi] * s16 + moff
                plsc.store_scatter(z_v, [bv, kv], zfin, mask=msk)

        def tile_body(gg, carry):
            t0 = 2 * gg
            h0 = start(t0, 0, rs0)
            h1 = start(t0 + 1, 1, rs1)
            h0.wait()
            compute_tile(t0, 0)
            h1.wait()
            compute_tile(t0 + 1, 1)
            return carry

        lax.fori_loop(0, TILES // 2, tile_body, 0)
        pltpu.sync_copy(z_v, z_h.at[pl.ds(base, C)])

    return sc_kernel(context, target, emb, node_w, paths, enc)


def _tc_loss_sum(z):
    n = z.size
    cols = 1024
    z2 = z.reshape(n // cols, cols)

    def body(z_ref, o_ref):
        x = z_ref[...]
        ll = jnp.minimum(x, 0.0) - jnp.log1p(jnp.exp(-jnp.abs(x)))
        o_ref[0, 0] = jnp.sum(ll)

    out = pl.pallas_call(
        body,
        out_shape=jax.ShapeDtypeStruct((1, 1), jnp.float32),
        out_specs=pl.BlockSpec(memory_space=pltpu.SMEM),
    )(z2)
    return out[0, 0]


def kernel(context, target, emb, node_w, paths, signs, mask):
    K = paths.shape[1]
    KP = -(-K // LANES) * LANES
    enc = signs + 3.0 * (1.0 - mask)
    pad = ((0, 0), (0, KP - K))
    paths_p = jnp.pad(paths, pad)
    enc_p = jnp.pad(enc, pad, constant_values=3.0)
    z = _sc_logits(context, target, emb, node_w, paths_p, enc_p, K)
    total = _tc_loss_sum(z)
    return -total / context.shape[0]


# speedup vs baseline: 1.6134x; 1.0004x over previous
import functools

import jax
import jax.numpy as jnp
from jax import lax
from jax.experimental import pallas as pl
from jax.experimental.pallas import tpu as pltpu
from jax.experimental.pallas import tpu_sc as plsc

NC = 2
NS = 16
NW = NC * NS
LANES = 16


def _sc_logits(context, target, emb, node_w, paths, enc, K):
    B = context.shape[0]
    V, D = emb.shape
    KP = paths.shape[1]
    NCH = D // LANES
    assert B % NW == 0 and D % LANES == 0
    C = B // NW
    GCH = 128
    assert C % GCH == 0
    IPT = 4
    assert C % (2 * IPT) == 0
    TILES = C // IPT
    NIDX = IPT * K
    assert NIDX <= 128
    TSTR = (NIDX + 7) // 8 * 8
    Q = C * K
    assert Q % LANES == 0
    NGRP = (NIDX + LANES - 1) // LANES
    TAIL = NIDX - (NGRP - 1) * LANES

    mesh = plsc.VectorSubcoreMesh(core_axis_name="c", subcore_axis_name="s")

    @functools.partial(
        pl.kernel,
        out_type=jax.ShapeDtypeStruct((B, K), jnp.float32),
        mesh=mesh,
        scratch_types=[
            pltpu.VMEM((C,), jnp.int32),
            pltpu.VMEM((C,), jnp.int32),
            pltpu.VMEM((C, D), jnp.float32),
            pltpu.VMEM((C, KP), jnp.int32),
            pltpu.VMEM((C, KP), jnp.float32),
            pltpu.VMEM((TILES, TSTR), jnp.int32),
            pltpu.VMEM((2, TSTR, D), jnp.float32),
            pltpu.VMEM((C, K), jnp.float32),
            pltpu.SemaphoreType.DMA,
            pltpu.SemaphoreType.DMA,
            pltpu.SemaphoreType.DMA,
        ],
        compiler_params=pltpu.CompilerParams(
            use_tc_tiling_on_sc=False,
            needs_layout_passes=False,
        ),
    )
    def sc_kernel(ctx_h, tgt_h, emb_h, nw_h, p_h, e_h, z_h,
                  ctx_v, tgt_v, inp_v, p_v, e_v, idx_b, w_r, z_v,
                  gsem, rs0, rs1):
        wid = lax.axis_index("s") * NC + lax.axis_index("c")
        base = wid * C
        pltpu.sync_copy(ctx_h.at[pl.ds(base, C)], ctx_v)
        pltpu.sync_copy(tgt_h.at[pl.ds(base, C)], tgt_v)

        lane = jnp.arange(LANES, dtype=jnp.int32)

        for g in range(C // GCH):
            sl = pl.ds(g * GCH, GCH)
            pltpu.async_copy(p_h.at[tgt_v.at[sl]], p_v.at[sl], gsem).wait()
            pltpu.async_copy(emb_h.at[ctx_v.at[sl]], inp_v.at[sl], gsem).wait()
            pltpu.async_copy(e_h.at[tgt_v.at[sl]], e_v.at[sl], gsem).wait()

        def flat_body(v, carry):
            jj = v * LANES + lane
            bb = jj // K
            kk = jj - bb * K
            vals = plsc.load_gather(p_v, [bb, kk])
            tt = jj // NIDX
            off = jj - tt * NIDX
            plsc.store_scatter(idx_b, [tt, off], vals)
            return carry

        NPAD = TSTR - NIDX
        zeros16 = jnp.zeros((LANES,), jnp.int32)

        def pad_body(v, carry):
            pp = v * LANES + lane
            tt = pp // NPAD
            rr = pp - tt * NPAD
            plsc.store_scatter(idx_b, [tt, NIDX + rr], zeros16)
            return carry

        lax.fori_loop(0, Q // LANES, flat_body, 0)
        lax.fori_loop(0, TILES * NPAD // LANES, pad_body, 0)

        def start(t, buf, sem):
            return pltpu.async_copy(
                nw_h.at[idx_b.at[t]], w_r.at[buf], sem)

        def compute_tile(t, half):
            zgs = [jnp.zeros((LANES,), jnp.float32) for _ in range(NGRP)]
            for ii in range(IPT):
                b_loc = t * IPT + ii
                inps = [inp_v[b_loc, pl.ds(c * LANES, LANES)] for c in range(NCH)]
                for kk in range(K):
                    jl = ii * K + kk
                    acc = w_r[half, jl, pl.ds(0, LANES)] * inps[0]
                    for c in range(1, NCH):
                        acc = acc + w_r[half, jl, pl.ds(c * LANES, LANES)] * inps[c]
                    rsum = jnp.sum(acc)
                    gi, pos = jl // LANES, jl % LANES
                    zgs[gi] = jnp.where(lane == pos, rsum, zgs[gi])
            for gi in range(NGRP):
                jj = t * NIDX + gi * LANES + lane
                jj = jnp.minimum(jj, Q - 1)
                bv = jj // K
                kv = jj - bv * K
                msk = (lane < TAIL) if gi == NGRP - 1 else None
                e16 = plsc.load_gather(e_v, [bv, kv], mask=msk)
                s16 = jnp.where(e16 > 2.0, 0.0, e16)
                moff = jnp.where(e16 > 2.0, 30.0, 0.0)
                zfin = zgs[gi] * s16 + moff
                plsc.store_scatter(z_v, [bv, kv], zfin, mask=msk)

        def wait_tile(t, buf, sem):
            pltpu.make_async_copy(
                nw_h.at[idx_b.at[t]], w_r.at[buf], sem).wait()

        start(0, 0, rs0)

        def tile_body(gg, carry):
            t0 = 2 * gg
            start(t0 + 1, 1, rs1)
            wait_tile(t0, 0, rs0)
            compute_tile(t0, 0)
            start(t0 + 2, 0, rs0)
            wait_tile(t0 + 1, 1, rs1)
            compute_tile(t0 + 1, 1)
            return carry

        lax.fori_loop(0, TILES // 2 - 1, tile_body, 0)
        start(TILES - 1, 1, rs1)
        wait_tile(TILES - 2, 0, rs0)
        compute_tile(TILES - 2, 0)
        wait_tile(TILES - 1, 1, rs1)
        compute_tile(TILES - 1, 1)
        pltpu.sync_copy(z_v, z_h.at[pl.ds(base, C)])

    return sc_kernel(context, target, emb, node_w, paths, enc)


def _tc_loss_sum(z):
    n = z.size
    cols = 1024
    z2 = z.reshape(n // cols, cols)

    def body(z_ref, o_ref):
        x = z_ref[...]
        ll = jnp.minimum(x, 0.0) - jnp.log1p(jnp.exp(-jnp.abs(x)))
        o_ref[0, 0] = jnp.sum(ll)

    out = pl.pallas_call(
        body,
        out_shape=jax.ShapeDtypeStruct((1, 1), jnp.float32),
        out_specs=pl.BlockSpec(memory_space=pltpu.SMEM),
    )(z2)
    return out[0, 0]


def kernel(context, target, emb, node_w, paths, signs, mask):
    K = paths.shape[1]
    KP = -(-K // LANES) * LANES
    enc = signs + 3.0 * (1.0 - mask)
    pad = ((0, 0), (0, KP - K))
    paths_p = jnp.pad(paths, pad)
    enc_p = jnp.pad(enc, pad, constant_values=3.0)
    z = _sc_logits(context, target, emb, node_w, paths_p, enc_p, K)
    total = _tc_loss_sum(z)
    return -total / context.shape[0]
